# Initial kernel scaffold; baseline (speedup 1.0000x reference)
import jax
import jax.numpy as jnp
from jax.experimental import pallas as pl


def kernel(x, expert_indices, expert_weights, gate_proj, up_proj, down_proj):
    raise NotImplementedError("write your pallas kernel here")


# speedup vs baseline: 1.0780x; 1.0780x over previous
import functools

import jax
import jax.numpy as jnp
from jax.experimental import pallas as pl

_HIDDEN = 2048
_INTER = 5632
_TI = 512


def _moe_body(idx_ref, w_ref, xt_ref, g_ref, u_ref, d_ref, out_ref):
    e = pl.program_id(0)
    i = pl.program_id(1)

    @pl.when(jnp.logical_and(e == 0, i == 0))
    def _init():
        out_ref[...] = jnp.zeros_like(out_ref)

    xt = xt_ref[...]
    g = jax.lax.dot_general(g_ref[0], xt, (((1,), (0,)), ((), ())),
                            preferred_element_type=jnp.float32)
    u = jax.lax.dot_general(u_ref[0], xt, (((1,), (0,)), ((), ())),
                            preferred_element_type=jnp.float32)
    h = (g * jax.nn.sigmoid(g)) * u

    ce = jnp.sum(jnp.where(idx_ref[...] == e, w_ref[...], 0.0), axis=1)
    h = h * ce[None, :]

    out_ref[...] += jax.lax.dot_general(d_ref[0], h, (((1,), (0,)), ((), ())),
                                        preferred_element_type=jnp.float32)


@functools.partial(jax.jit, static_argnames=())
def kernel(x, expert_indices, expert_weights, gate_proj, up_proj, down_proj):
    batch, seq_len, hidden = x.shape
    num_experts = gate_proj.shape[0]
    inter = gate_proj.shape[1]
    top_k = expert_indices.shape[-1]
    num_tokens = batch * seq_len

    xt = x.reshape(num_tokens, hidden).T
    idx = expert_indices.reshape(num_tokens, top_k)
    w = expert_weights.reshape(num_tokens, top_k)

    n_i = inter // _TI
    grid = (num_experts, n_i)

    out_t = pl.pallas_call(
        _moe_body,
        grid=grid,
        in_specs=[
            pl.BlockSpec((num_tokens, top_k), lambda e, i: (0, 0)),
            pl.BlockSpec((num_tokens, top_k), lambda e, i: (0, 0)),
            pl.BlockSpec((hidden, num_tokens), lambda e, i: (0, 0)),
            pl.BlockSpec((1, _TI, hidden), lambda e, i: (e, i, 0)),
            pl.BlockSpec((1, _TI, hidden), lambda e, i: (e, i, 0)),
            pl.BlockSpec((1, hidden, _TI), lambda e, i: (e, 0, i)),
        ],
        out_specs=pl.BlockSpec((hidden, num_tokens), lambda e, i: (0, 0)),
        out_shape=jax.ShapeDtypeStruct((hidden, num_tokens), jnp.float32),
    )(idx, w, xt, gate_proj, up_proj, down_proj)

    return out_t.T.reshape(batch, seq_len, hidden)
